# baseline (speedup 1.0000x reference)
import jax
import jax.numpy as jnp
from jax import lax
from jax.experimental import pallas as pl
from jax.experimental.pallas import tpu as pltpu
from jax.experimental.pallas import tpu_sc as plsc

N = 10000
E = 320000
D = 128
NC, NS = 2, 16
NW = NC * NS
EPW = E // NW
CH = 80
NCH = EPW // CH
NBUF = 4
CW = 16
CPB = 624
MB = 1000
f32 = jnp.float32


def _make_sc_propagate():
    mesh = plsc.VectorSubcoreMesh(core_axis_name="c", subcore_axis_name="s",
                                  num_cores=NC, num_subcores=NS)
    out_type = jax.ShapeDtypeStruct((NC * N, D), f32)
    scratch = (
        [pltpu.VMEM((2, CH), jnp.int32) for _ in range(NBUF)]
        + [pltpu.VMEM((CH,), f32) for _ in range(NBUF)]
        + [pltpu.VMEM((CH, D), f32) for _ in range(NBUF)]
        + [pltpu.VMEM_SHARED((N, D), f32)]
        + [pltpu.SemaphoreType.DMA for _ in range(3 * NBUF)]
    )

    def body(h, srcf, dstf, ewf, s_out, *rest):
        ebufs = rest[:NBUF]
        wbufs = rest[NBUF:2 * NBUF]
        rows = rest[2 * NBUF:3 * NBUF]
        acc = rest[3 * NBUF]
        sems = rest[3 * NBUF + 1:]
        esems = sems[:NBUF]
        gsems = sems[NBUF:2 * NBUF]
        ssems = sems[2 * NBUF:3 * NBUF]

        cid = lax.axis_index("c")
        sid = lax.axis_index("s")
        wid = cid * NS + sid
        ebase = wid * EPW
        z16 = jnp.zeros((16,), f32)

        def load_blocks(c, b):
            eo = pl.multiple_of(ebase + c * CH, 8)
            pltpu.async_copy(srcf.at[pl.ds(eo, CH)], ebufs[b].at[0], esems[b])
            pltpu.async_copy(dstf.at[pl.ds(eo, CH)], ebufs[b].at[1], esems[b])
            pltpu.async_copy(ewf.at[pl.ds(eo, CH)], wbufs[b], esems[b])

        def wait_blocks(c, b):
            eo = pl.multiple_of(ebase + c * CH, 8)
            pltpu.make_async_copy(srcf.at[pl.ds(eo, CH)], ebufs[b].at[0],
                                  esems[b]).wait()
            pltpu.make_async_copy(dstf.at[pl.ds(eo, CH)], ebufs[b].at[1],
                                  esems[b]).wait()
            pltpu.make_async_copy(ewf.at[pl.ds(eo, CH)], wbufs[b],
                                  esems[b]).wait()

        load_blocks(0, 0)
        load_blocks(1, 1)

        @pl.loop(0, CH)
        def _zrow(r):
            for j in range(D // 16):
                rows[0][r, pl.ds(j * 16, 16)] = z16

        @pl.when(sid < NS - 1)
        def _():
            zb = pl.multiple_of(sid * CPB, 8)
            for k in range(7):
                pltpu.sync_copy(rows[0], acc.at[pl.ds(zb + k * CH, CH)])
            pltpu.sync_copy(rows[0].at[pl.ds(0, 64)],
                            acc.at[pl.ds(zb + 7 * CH, 64)])

        @pl.when(sid == NS - 1)
        def _():
            zb = (NS - 1) * CPB
            for k in range(8):
                pltpu.sync_copy(rows[0], acc.at[pl.ds(zb + k * CH, CH)])

        plsc.subcore_barrier()

        wait_blocks(0, 0)
        pltpu.async_copy(h.at[ebufs[0].at[0]], rows[0], gsems[0])

        def chunk_step(c, b, in_loop):
            bn = (b + 1) % NBUF
            b2 = (b + 2) % NBUF
            if in_loop:
                @pl.when((c >= 2) & (c + 2 < NCH))
                def _():
                    pltpu.make_async_copy(
                        rows[b2], acc.at[ebufs[b2].at[1]], ssems[b2]).wait()

                @pl.when(c + 2 < NCH)
                def _():
                    load_blocks(c + 2, b2)

                wait_blocks(c + 1, bn)
                pltpu.async_copy(h.at[ebufs[bn].at[0]], rows[bn], gsems[bn])
            pltpu.make_async_copy(h.at[ebufs[b].at[0]], rows[b],
                                  gsems[b]).wait()
            buf = rows[b]
            eb = ebufs[b]
            wb = wbufs[b]

            @plsc.parallel_loop(0, CH // 16, unroll=2)
            def _scale(g):
                sgf = 0.5 + 0.5 * wb[pl.ds(g * 16, 16)]
                for l in range(16):
                    sv = jnp.full((16,), sgf[l], f32)
                    r = g * 16 + l
                    for j in range(D // 16):
                        buf[r, pl.ds(j * 16, 16)] = (
                            buf[r, pl.ds(j * 16, 16)] * sv)

            pltpu.async_copy(buf, acc.at[eb.at[1]], ssems[b], add=True)

        @pl.loop(0, NCH - 1, step=NBUF)
        def _chunks(c0):
            for b in range(NBUF):
                chunk_step(c0 + b, b, True)

        chunk_step(NCH - 1, (NCH - 1) % NBUF, False)

        for b in range(NBUF):
            pltpu.make_async_copy(rows[b], acc.at[ebufs[b].at[1]],
                                  ssems[b]).wait()

        plsc.subcore_barrier()

        @pl.when(sid < NS - 1)
        def _():
            ob = pl.multiple_of(sid * CPB, 8)
            oo = pl.multiple_of(cid * N + sid * CPB, 8)
            pltpu.sync_copy(acc.at[pl.ds(ob, CPB)],
                            s_out.at[pl.ds(oo, CPB)])

        @pl.when(sid == NS - 1)
        def _():
            ob = (NS - 1) * CPB
            oo = pl.multiple_of(cid * N + ob, 8)
            pltpu.sync_copy(acc.at[pl.ds(ob, N - ob)],
                            s_out.at[pl.ds(oo, N - ob)])

    return pl.kernel(body, out_type=out_type, mesh=mesh, scratch_types=scratch)


def _mm(x, w, b):
    n, din = x.shape
    dout = w.shape[1]

    def bodyfn(x_ref, w_ref, b_ref, o_ref):
        o_ref[:] = jnp.dot(x_ref[:], w_ref[:],
                           preferred_element_type=f32) + b_ref[:]

    return pl.pallas_call(
        bodyfn,
        grid=(n // MB,),
        in_specs=[pl.BlockSpec((MB, din), lambda i: (i, 0)),
                  pl.BlockSpec((din, dout), lambda i: (0, 0)),
                  pl.BlockSpec((1, dout), lambda i: (0, 0))],
        out_specs=pl.BlockSpec((MB, dout), lambda i: (i, 0)),
        out_shape=jax.ShapeDtypeStruct((n, dout), f32),
    )(x, w, b)


def _update(S, C, x, awa, awb, ab):
    NBLK = N // MB

    def bodyfn(p0r, p1r, c0r, c1r, xr, ar, br, abr, o_ref):
        cnt = c0r[:, 0:1] + c1r[:, 0:1]
        recip = 1.0 / jnp.maximum(cnt, 1.0)
        aggr = (p0r[:] + p1r[:]) * recip
        o_ref[:] = jnp.maximum(
            jnp.dot(aggr, ar[:], preferred_element_type=f32)
            + jnp.dot(xr[:], br[:], preferred_element_type=f32) + abr[:], 0.0)

    return pl.pallas_call(
        bodyfn,
        grid=(NBLK,),
        in_specs=[pl.BlockSpec((MB, D), lambda i: (i, 0)),
                  pl.BlockSpec((MB, D), lambda i: (i + NBLK, 0)),
                  pl.BlockSpec((MB, D), lambda i: (i, 0)),
                  pl.BlockSpec((MB, D), lambda i: (i + NBLK, 0)),
                  pl.BlockSpec((MB, D), lambda i: (i, 0)),
                  pl.BlockSpec((D, D), lambda i: (0, 0)),
                  pl.BlockSpec((D, D), lambda i: (0, 0)),
                  pl.BlockSpec((1, D), lambda i: (0, 0))],
        out_specs=pl.BlockSpec((MB, D), lambda i: (i, 0)),
        out_shape=jax.ShapeDtypeStruct((N, D), f32),
    )(S, S, C, C, x, awa, awb, ab)


def _classifier(h, w0, b0, w1, b1):
    grid = N // MB

    def bodyfn(h_ref, w0r, b0r, w1r, b1r, o_ref, accr):
        i = pl.program_id(0)

        @pl.when(i == 0)
        def _():
            accr[:] = jnp.zeros((1, D), f32)

        accr[:] = accr[:] + jnp.sum(h_ref[:], axis=0, keepdims=True)

        @pl.when(i == grid - 1)
        def _():
            pooled = accr[:] * (1.0 / N)
            z = jnp.maximum(
                jnp.dot(pooled, w0r[:], preferred_element_type=f32) + b0r[:],
                0.0)
            o_ref[:] = jnp.dot(z, w1r[:], preferred_element_type=f32) + b1r[:]

    out = pl.pallas_call(
        bodyfn,
        grid=(grid,),
        in_specs=[pl.BlockSpec((MB, D), lambda i: (i, 0)),
                  pl.BlockSpec((D, D), lambda i: (0, 0)),
                  pl.BlockSpec((1, D), lambda i: (0, 0)),
                  pl.BlockSpec((D, 16), lambda i: (0, 0)),
                  pl.BlockSpec((1, 16), lambda i: (0, 0))],
        out_specs=pl.BlockSpec((1, 16), lambda i: (0, 0)),
        out_shape=jax.ShapeDtypeStruct((1, 16), f32),
        scratch_shapes=[pltpu.VMEM((1, D), f32)],
    )(h, w0, b0, w1, b1)
    return out.reshape(16)


def kernel(x, edge_index, edge_weight,
           lin_W0, lin_b0, agg_W0, agg_b0,
           lin_W1, lin_b1, agg_W1, agg_b1,
           lin_W2, lin_b2, agg_W2, agg_b2,
           cls_W0, cls_b0, cls_W1, cls_b1):
    srcf = edge_index[1]
    dstf = edge_index[0]

    sc_prop = _make_sc_propagate()

    C = sc_prop(jnp.ones((N, D), f32), srcf, dstf, jnp.ones((E,), f32))

    layers = ((lin_W0, lin_b0, agg_W0, agg_b0),
              (lin_W1, lin_b1, agg_W1, agg_b1),
              (lin_W2, lin_b2, agg_W2, agg_b2))

    xcur = x
    for (W, b, AW, Ab) in layers:
        h = _mm(xcur, W, b.reshape(1, D))
        S = sc_prop(h, srcf, dstf, edge_weight)
        xcur = _update(S, C, xcur, AW[:D], AW[D:], Ab.reshape(1, D))

    return _classifier(xcur, cls_W0, cls_b0.reshape(1, D),
                       cls_W1, cls_b1.reshape(1, 16))


# speedup vs baseline: 1.1659x; 1.1659x over previous
import jax
import jax.numpy as jnp
from jax import lax
from jax.experimental import pallas as pl
from jax.experimental.pallas import tpu as pltpu
from jax.experimental.pallas import tpu_sc as plsc

N = 10000
E = 320000
D = 128
NC, NS = 2, 16
NW = NC * NS
EPW = E // NW
CH = 80
NCH = EPW // CH
NBUF = 4
CW = 16
CPB = 624
MB = 1000
f32 = jnp.float32


def _make_sc_propagate():
    mesh = plsc.VectorSubcoreMesh(core_axis_name="c", subcore_axis_name="s",
                                  num_cores=NC, num_subcores=NS)
    out_type = jax.ShapeDtypeStruct((NC * N, D), f32)
    scratch = (
        [pltpu.VMEM((2, CH), jnp.int32) for _ in range(NBUF)]
        + [pltpu.VMEM((CH,), f32) for _ in range(NBUF)]
        + [pltpu.VMEM((CH, D), f32) for _ in range(NBUF)]
        + [pltpu.VMEM_SHARED((N, D), f32)]
        + [pltpu.SemaphoreType.DMA for _ in range(3 * NBUF)]
    )

    def body(h, srcf, dstf, ewf, s_out, *rest):
        ebufs = rest[:NBUF]
        wbufs = rest[NBUF:2 * NBUF]
        rows = rest[2 * NBUF:3 * NBUF]
        acc = rest[3 * NBUF]
        sems = rest[3 * NBUF + 1:]
        esems = sems[:NBUF]
        gsems = sems[NBUF:2 * NBUF]
        ssems = sems[2 * NBUF:3 * NBUF]

        cid = lax.axis_index("c")
        sid = lax.axis_index("s")
        wid = cid * NS + sid
        ebase = wid * EPW
        z16 = jnp.zeros((16,), f32)

        def load_blocks(c, b):
            eo = pl.multiple_of(ebase + c * CH, 8)
            pltpu.async_copy(srcf.at[pl.ds(eo, CH)], ebufs[b].at[0], esems[b])
            pltpu.async_copy(dstf.at[pl.ds(eo, CH)], ebufs[b].at[1], esems[b])
            pltpu.async_copy(ewf.at[pl.ds(eo, CH)], wbufs[b], esems[b])

        def wait_blocks(c, b):
            eo = pl.multiple_of(ebase + c * CH, 8)
            pltpu.make_async_copy(srcf.at[pl.ds(eo, CH)], ebufs[b].at[0],
                                  esems[b]).wait()
            pltpu.make_async_copy(dstf.at[pl.ds(eo, CH)], ebufs[b].at[1],
                                  esems[b]).wait()
            pltpu.make_async_copy(ewf.at[pl.ds(eo, CH)], wbufs[b],
                                  esems[b]).wait()

        load_blocks(0, 0)
        load_blocks(1, 1)

        @pl.loop(0, CH)
        def _zrow(r):
            for j in range(D // 16):
                rows[0][r, pl.ds(j * 16, 16)] = z16

        @pl.when(sid < NS - 1)
        def _():
            zb = pl.multiple_of(sid * CPB, 8)
            for k in range(7):
                pltpu.sync_copy(rows[0], acc.at[pl.ds(zb + k * CH, CH)])
            pltpu.sync_copy(rows[0].at[pl.ds(0, 64)],
                            acc.at[pl.ds(zb + 7 * CH, 64)])

        @pl.when(sid == NS - 1)
        def _():
            zb = (NS - 1) * CPB
            for k in range(8):
                pltpu.sync_copy(rows[0], acc.at[pl.ds(zb + k * CH, CH)])

        plsc.subcore_barrier()

        wait_blocks(0, 0)
        pltpu.async_copy(h.at[ebufs[0].at[0]], rows[0], gsems[0])

        def chunk_step(c, b, in_loop):
            bn = (b + 1) % NBUF
            b2 = (b + 2) % NBUF
            if in_loop:
                @pl.when((c >= 2) & (c + 2 < NCH))
                def _():
                    pltpu.make_async_copy(
                        rows[b2], acc.at[ebufs[b2].at[1]], ssems[b2]).wait()

                @pl.when(c + 2 < NCH)
                def _():
                    load_blocks(c + 2, b2)

                wait_blocks(c + 1, bn)
                pltpu.async_copy(h.at[ebufs[bn].at[0]], rows[bn], gsems[bn])
            pltpu.make_async_copy(h.at[ebufs[b].at[0]], rows[b],
                                  gsems[b]).wait()
            buf = rows[b]
            eb = ebufs[b]
            wb = wbufs[b]

            @pl.loop(0, CH // 16)
            def _scale(g):
                sgf = 0.5 + 0.5 * wb[pl.ds(g * 16, 16)]
                for l in range(16):
                    sv = jnp.full((16,), sgf[l], f32)
                    r = g * 16 + l
                    for j in range(D // 16):
                        buf[r, pl.ds(j * 16, 16)] = (
                            buf[r, pl.ds(j * 16, 16)] * sv)

            pltpu.async_copy(buf, acc.at[eb.at[1]], ssems[b], add=True)

        @pl.loop(0, NCH - 1, step=NBUF)
        def _chunks(c0):
            for b in range(NBUF):
                chunk_step(c0 + b, b, True)

        chunk_step(NCH - 1, (NCH - 1) % NBUF, False)

        for b in range(NBUF):
            pltpu.make_async_copy(rows[b], acc.at[ebufs[b].at[1]],
                                  ssems[b]).wait()

        plsc.subcore_barrier()

        @pl.when(sid < NS - 1)
        def _():
            ob = pl.multiple_of(sid * CPB, 8)
            oo = pl.multiple_of(cid * N + sid * CPB, 8)
            pltpu.sync_copy(acc.at[pl.ds(ob, CPB)],
                            s_out.at[pl.ds(oo, CPB)])

        @pl.when(sid == NS - 1)
        def _():
            ob = (NS - 1) * CPB
            oo = pl.multiple_of(cid * N + ob, 8)
            pltpu.sync_copy(acc.at[pl.ds(ob, N - ob)],
                            s_out.at[pl.ds(oo, N - ob)])

    return pl.kernel(body, out_type=out_type, mesh=mesh, scratch_types=scratch)


def _mm(x, w, b):
    n, din = x.shape
    dout = w.shape[1]

    def bodyfn(x_ref, w_ref, b_ref, o_ref):
        o_ref[:] = jnp.dot(x_ref[:], w_ref[:],
                           preferred_element_type=f32) + b_ref[:]

    return pl.pallas_call(
        bodyfn,
        grid=(n // MB,),
        in_specs=[pl.BlockSpec((MB, din), lambda i: (i, 0)),
                  pl.BlockSpec((din, dout), lambda i: (0, 0)),
                  pl.BlockSpec((1, dout), lambda i: (0, 0))],
        out_specs=pl.BlockSpec((MB, dout), lambda i: (i, 0)),
        out_shape=jax.ShapeDtypeStruct((n, dout), f32),
    )(x, w, b)


def _update(S, C, x, awa, awb, ab):
    NBLK = N // MB

    def bodyfn(p0r, p1r, c0r, c1r, xr, ar, br, abr, o_ref):
        cnt = c0r[:, 0:1] + c1r[:, 0:1]
        recip = 1.0 / jnp.maximum(cnt, 1.0)
        aggr = (p0r[:] + p1r[:]) * recip
        o_ref[:] = jnp.maximum(
            jnp.dot(aggr, ar[:], preferred_element_type=f32)
            + jnp.dot(xr[:], br[:], preferred_element_type=f32) + abr[:], 0.0)

    return pl.pallas_call(
        bodyfn,
        grid=(NBLK,),
        in_specs=[pl.BlockSpec((MB, D), lambda i: (i, 0)),
                  pl.BlockSpec((MB, D), lambda i: (i + NBLK, 0)),
                  pl.BlockSpec((MB, D), lambda i: (i, 0)),
                  pl.BlockSpec((MB, D), lambda i: (i + NBLK, 0)),
                  pl.BlockSpec((MB, D), lambda i: (i, 0)),
                  pl.BlockSpec((D, D), lambda i: (0, 0)),
                  pl.BlockSpec((D, D), lambda i: (0, 0)),
                  pl.BlockSpec((1, D), lambda i: (0, 0))],
        out_specs=pl.BlockSpec((MB, D), lambda i: (i, 0)),
        out_shape=jax.ShapeDtypeStruct((N, D), f32),
    )(S, S, C, C, x, awa, awb, ab)


def _update_mm(S, C, x, awa, awb, ab, wn, bn):
    NBLK = N // MB

    def bodyfn(p0r, p1r, c0r, c1r, xr, ar, br, abr, wnr, bnr, xo, ho):
        cnt = c0r[:, 0:1] + c1r[:, 0:1]
        recip = 1.0 / jnp.maximum(cnt, 1.0)
        aggr = (p0r[:] + p1r[:]) * recip
        xn = jnp.maximum(
            jnp.dot(aggr, ar[:], preferred_element_type=f32)
            + jnp.dot(xr[:], br[:], preferred_element_type=f32) + abr[:], 0.0)
        xo[:] = xn
        ho[:] = jnp.dot(xn, wnr[:], preferred_element_type=f32) + bnr[:]

    return pl.pallas_call(
        bodyfn,
        grid=(NBLK,),
        in_specs=[pl.BlockSpec((MB, D), lambda i: (i, 0)),
                  pl.BlockSpec((MB, D), lambda i: (i + NBLK, 0)),
                  pl.BlockSpec((MB, D), lambda i: (i, 0)),
                  pl.BlockSpec((MB, D), lambda i: (i + NBLK, 0)),
                  pl.BlockSpec((MB, D), lambda i: (i, 0)),
                  pl.BlockSpec((D, D), lambda i: (0, 0)),
                  pl.BlockSpec((D, D), lambda i: (0, 0)),
                  pl.BlockSpec((1, D), lambda i: (0, 0)),
                  pl.BlockSpec((D, D), lambda i: (0, 0)),
                  pl.BlockSpec((1, D), lambda i: (0, 0))],
        out_specs=[pl.BlockSpec((MB, D), lambda i: (i, 0)),
                   pl.BlockSpec((MB, D), lambda i: (i, 0))],
        out_shape=[jax.ShapeDtypeStruct((N, D), f32),
                   jax.ShapeDtypeStruct((N, D), f32)],
    )(S, S, C, C, x, awa, awb, ab, wn, bn)


def _update_cls(S, C, x, awa, awb, ab, w0, b0, w1, b1):
    NBLK = N // MB

    def bodyfn(p0r, p1r, c0r, c1r, xr, ar, br, abr,
               w0r, b0r, w1r, b1r, o_ref, accr):
        i = pl.program_id(0)
        cnt = c0r[:, 0:1] + c1r[:, 0:1]
        recip = 1.0 / jnp.maximum(cnt, 1.0)
        aggr = (p0r[:] + p1r[:]) * recip
        xn = jnp.maximum(
            jnp.dot(aggr, ar[:], preferred_element_type=f32)
            + jnp.dot(xr[:], br[:], preferred_element_type=f32) + abr[:], 0.0)

        @pl.when(i == 0)
        def _():
            accr[:] = jnp.zeros((1, D), f32)

        accr[:] = accr[:] + jnp.sum(xn, axis=0, keepdims=True)

        @pl.when(i == NBLK - 1)
        def _():
            pooled = accr[:] * (1.0 / N)
            z = jnp.maximum(
                jnp.dot(pooled, w0r[:], preferred_element_type=f32) + b0r[:],
                0.0)
            o_ref[:] = jnp.dot(z, w1r[:], preferred_element_type=f32) + b1r[:]

    out = pl.pallas_call(
        bodyfn,
        grid=(NBLK,),
        in_specs=[pl.BlockSpec((MB, D), lambda i: (i, 0)),
                  pl.BlockSpec((MB, D), lambda i: (i + NBLK, 0)),
                  pl.BlockSpec((MB, D), lambda i: (i, 0)),
                  pl.BlockSpec((MB, D), lambda i: (i + NBLK, 0)),
                  pl.BlockSpec((MB, D), lambda i: (i, 0)),
                  pl.BlockSpec((D, D), lambda i: (0, 0)),
                  pl.BlockSpec((D, D), lambda i: (0, 0)),
                  pl.BlockSpec((1, D), lambda i: (0, 0)),
                  pl.BlockSpec((D, D), lambda i: (0, 0)),
                  pl.BlockSpec((1, D), lambda i: (0, 0)),
                  pl.BlockSpec((D, 16), lambda i: (0, 0)),
                  pl.BlockSpec((1, 16), lambda i: (0, 0))],
        out_specs=pl.BlockSpec((1, 16), lambda i: (0, 0)),
        out_shape=jax.ShapeDtypeStruct((1, 16), f32),
        scratch_shapes=[pltpu.VMEM((1, D), f32)],
    )(S, S, C, C, x, awa, awb, ab, w0, b0, w1, b1)
    return out.reshape(16)


def _classifier(h, w0, b0, w1, b1):
    grid = N // MB

    def bodyfn(h_ref, w0r, b0r, w1r, b1r, o_ref, accr):
        i = pl.program_id(0)

        @pl.when(i == 0)
        def _():
            accr[:] = jnp.zeros((1, D), f32)

        accr[:] = accr[:] + jnp.sum(h_ref[:], axis=0, keepdims=True)

        @pl.when(i == grid - 1)
        def _():
            pooled = accr[:] * (1.0 / N)
            z = jnp.maximum(
                jnp.dot(pooled, w0r[:], preferred_element_type=f32) + b0r[:],
                0.0)
            o_ref[:] = jnp.dot(z, w1r[:], preferred_element_type=f32) + b1r[:]

    out = pl.pallas_call(
        bodyfn,
        grid=(grid,),
        in_specs=[pl.BlockSpec((MB, D), lambda i: (i, 0)),
                  pl.BlockSpec((D, D), lambda i: (0, 0)),
                  pl.BlockSpec((1, D), lambda i: (0, 0)),
                  pl.BlockSpec((D, 16), lambda i: (0, 0)),
                  pl.BlockSpec((1, 16), lambda i: (0, 0))],
        out_specs=pl.BlockSpec((1, 16), lambda i: (0, 0)),
        out_shape=jax.ShapeDtypeStruct((1, 16), f32),
        scratch_shapes=[pltpu.VMEM((1, D), f32)],
    )(h, w0, b0, w1, b1)
    return out.reshape(16)


def kernel(x, edge_index, edge_weight,
           lin_W0, lin_b0, agg_W0, agg_b0,
           lin_W1, lin_b1, agg_W1, agg_b1,
           lin_W2, lin_b2, agg_W2, agg_b2,
           cls_W0, cls_b0, cls_W1, cls_b1):
    srcf = edge_index[1]
    dstf = edge_index[0]

    sc_prop = _make_sc_propagate()

    C = sc_prop(jnp.ones((N, D), f32), srcf, dstf, jnp.ones((E,), f32))

    h = _mm(x, lin_W0, lin_b0.reshape(1, D))
    S = sc_prop(h, srcf, dstf, edge_weight)
    x1, h1 = _update_mm(S, C, x, agg_W0[:D], agg_W0[D:],
                        agg_b0.reshape(1, D), lin_W1, lin_b1.reshape(1, D))
    S = sc_prop(h1, srcf, dstf, edge_weight)
    x2, h2 = _update_mm(S, C, x1, agg_W1[:D], agg_W1[D:],
                        agg_b1.reshape(1, D), lin_W2, lin_b2.reshape(1, D))
    S = sc_prop(h2, srcf, dstf, edge_weight)
    return _update_cls(S, C, x2, agg_W2[:D], agg_W2[D:],
                       agg_b2.reshape(1, D), cls_W0, cls_b0.reshape(1, D),
                       cls_W1, cls_b1.reshape(1, 16))


# speedup vs baseline: 1.2950x; 1.1107x over previous
import jax
import jax.numpy as jnp
from jax import lax
from jax.experimental import pallas as pl
from jax.experimental.pallas import tpu as pltpu
from jax.experimental.pallas import tpu_sc as plsc

N = 10000
E = 320000
D = 128
NC, NS = 2, 16
NW = NC * NS
EPW = E // NW
CH = 80
NCH = EPW // CH
NBUF = 4
CW = 16
CPB = 624
MB = 1000
f32 = jnp.float32


def _make_sc_propagate(with_gather=True):
    mesh = plsc.VectorSubcoreMesh(core_axis_name="c", subcore_axis_name="s",
                                  num_cores=NC, num_subcores=NS)
    out_type = jax.ShapeDtypeStruct((NC * N, D), f32)
    scratch = (
        [pltpu.VMEM((2, CH), jnp.int32) for _ in range(NBUF)]
        + [pltpu.VMEM((CH,), f32) for _ in range(NBUF)]
        + [pltpu.VMEM((CH, D), f32) for _ in range(NBUF)]
        + [pltpu.VMEM_SHARED((N, D), f32)]
        + [pltpu.SemaphoreType.DMA for _ in range(3 * NBUF)]
    )

    def body(h, srcf, dstf, ewf, s_out, *rest):
        ebufs = rest[:NBUF]
        wbufs = rest[NBUF:2 * NBUF]
        rows = rest[2 * NBUF:3 * NBUF]
        acc = rest[3 * NBUF]
        sems = rest[3 * NBUF + 1:]
        esems = sems[:NBUF]
        gsems = sems[NBUF:2 * NBUF]
        ssems = sems[2 * NBUF:3 * NBUF]

        cid = lax.axis_index("c")
        sid = lax.axis_index("s")
        wid = cid * NS + sid
        ebase = wid * EPW
        z16 = jnp.zeros((16,), f32)

        def load_blocks(c, b):
            eo = pl.multiple_of(ebase + c * CH, 8)
            pltpu.async_copy(srcf.at[pl.ds(eo, CH)], ebufs[b].at[0], esems[b])
            pltpu.async_copy(dstf.at[pl.ds(eo, CH)], ebufs[b].at[1], esems[b])
            pltpu.async_copy(ewf.at[pl.ds(eo, CH)], wbufs[b], esems[b])

        def wait_blocks(c, b):
            eo = pl.multiple_of(ebase + c * CH, 8)
            pltpu.make_async_copy(srcf.at[pl.ds(eo, CH)], ebufs[b].at[0],
                                  esems[b]).wait()
            pltpu.make_async_copy(dstf.at[pl.ds(eo, CH)], ebufs[b].at[1],
                                  esems[b]).wait()
            pltpu.make_async_copy(ewf.at[pl.ds(eo, CH)], wbufs[b],
                                  esems[b]).wait()

        load_blocks(0, 0)
        load_blocks(1, 1)

        @pl.loop(0, CH)
        def _zrow(r):
            for j in range(D // 16):
                rows[0][r, pl.ds(j * 16, 16)] = z16

        @pl.when(sid < NS - 1)
        def _():
            zb = pl.multiple_of(sid * CPB, 8)
            for k in range(7):
                pltpu.sync_copy(rows[0], acc.at[pl.ds(zb + k * CH, CH)])
            pltpu.sync_copy(rows[0].at[pl.ds(0, 64)],
                            acc.at[pl.ds(zb + 7 * CH, 64)])

        @pl.when(sid == NS - 1)
        def _():
            zb = (NS - 1) * CPB
            for k in range(8):
                pltpu.sync_copy(rows[0], acc.at[pl.ds(zb + k * CH, CH)])

        plsc.subcore_barrier()

        wait_blocks(0, 0)
        if with_gather:
            pltpu.async_copy(h.at[ebufs[0].at[0]], rows[0], gsems[0])
        else:
            @pl.loop(0, CH)
            def _orow(r):
                for j in range(D // 16):
                    rows[0][r, pl.ds(j * 16, 16)] = jnp.ones((16,), f32)

        def chunk_step(c, b, in_loop):
            bn = (b + 1) % NBUF
            b2 = (b + 2) % NBUF
            if in_loop:
                @pl.when((c >= 2) & (c + 2 < NCH))
                def _():
                    pltpu.make_async_copy(
                        rows[b2], acc.at[ebufs[b2].at[1]], ssems[b2]).wait()

                @pl.when(c + 2 < NCH)
                def _():
                    load_blocks(c + 2, b2)

                wait_blocks(c + 1, bn)
                if with_gather:
                    pltpu.async_copy(h.at[ebufs[bn].at[0]], rows[bn],
                                     gsems[bn])
            eb = ebufs[b]
            if with_gather:
                pltpu.make_async_copy(h.at[ebufs[b].at[0]], rows[b],
                                      gsems[b]).wait()
                buf = rows[b]
                wb = wbufs[b]

                @pl.loop(0, CH // 16)
                def _scale(g):
                    sgf = 0.5 + 0.5 * wb[pl.ds(g * 16, 16)]
                    for l in range(16):
                        sv = jnp.full((16,), sgf[l], f32)
                        r = g * 16 + l
                        for j in range(D // 16):
                            buf[r, pl.ds(j * 16, 16)] = (
                                buf[r, pl.ds(j * 16, 16)] * sv)
            else:
                buf = rows[0]

            pltpu.async_copy(buf, acc.at[eb.at[1]], ssems[b], add=True)

        @pl.loop(0, NCH - 1, step=NBUF)
        def _chunks(c0):
            for b in range(NBUF):
                chunk_step(c0 + b, b, True)

        chunk_step(NCH - 1, (NCH - 1) % NBUF, False)

        for b in range(NBUF):
            sbuf = rows[b] if with_gather else rows[0]
            pltpu.make_async_copy(sbuf, acc.at[ebufs[b].at[1]],
                                  ssems[b]).wait()

        plsc.subcore_barrier()

        @pl.when(sid < NS - 1)
        def _():
            ob = pl.multiple_of(sid * CPB, 8)
            oo = pl.multiple_of(cid * N + sid * CPB, 8)
            pltpu.sync_copy(acc.at[pl.ds(ob, CPB)],
                            s_out.at[pl.ds(oo, CPB)])

        @pl.when(sid == NS - 1)
        def _():
            ob = (NS - 1) * CPB
            oo = pl.multiple_of(cid * N + ob, 8)
            pltpu.sync_copy(acc.at[pl.ds(ob, N - ob)],
                            s_out.at[pl.ds(oo, N - ob)])

    return pl.kernel(body, out_type=out_type, mesh=mesh, scratch_types=scratch)


def _mm(x, w, b):
    n, din = x.shape
    dout = w.shape[1]

    def bodyfn(x_ref, w_ref, b_ref, o_ref):
        o_ref[:] = jnp.dot(x_ref[:], w_ref[:],
                           preferred_element_type=f32) + b_ref[:]

    return pl.pallas_call(
        bodyfn,
        grid=(n // MB,),
        in_specs=[pl.BlockSpec((MB, din), lambda i: (i, 0)),
                  pl.BlockSpec((din, dout), lambda i: (0, 0)),
                  pl.BlockSpec((1, dout), lambda i: (0, 0))],
        out_specs=pl.BlockSpec((MB, dout), lambda i: (i, 0)),
        out_shape=jax.ShapeDtypeStruct((n, dout), f32),
    )(x, w, b)


def _update(S, C, x, awa, awb, ab):
    NBLK = N // MB

    def bodyfn(p0r, p1r, c0r, c1r, xr, ar, br, abr, o_ref):
        cnt = c0r[:, 0:1] + c1r[:, 0:1]
        recip = 1.0 / jnp.maximum(cnt, 1.0)
        aggr = (p0r[:] + p1r[:]) * recip
        o_ref[:] = jnp.maximum(
            jnp.dot(aggr, ar[:], preferred_element_type=f32)
            + jnp.dot(xr[:], br[:], preferred_element_type=f32) + abr[:], 0.0)

    return pl.pallas_call(
        bodyfn,
        grid=(NBLK,),
        in_specs=[pl.BlockSpec((MB, D), lambda i: (i, 0)),
                  pl.BlockSpec((MB, D), lambda i: (i + NBLK, 0)),
                  pl.BlockSpec((MB, D), lambda i: (i, 0)),
                  pl.BlockSpec((MB, D), lambda i: (i + NBLK, 0)),
                  pl.BlockSpec((MB, D), lambda i: (i, 0)),
                  pl.BlockSpec((D, D), lambda i: (0, 0)),
                  pl.BlockSpec((D, D), lambda i: (0, 0)),
                  pl.BlockSpec((1, D), lambda i: (0, 0))],
        out_specs=pl.BlockSpec((MB, D), lambda i: (i, 0)),
        out_shape=jax.ShapeDtypeStruct((N, D), f32),
    )(S, S, C, C, x, awa, awb, ab)


def _update_mm(S, C, x, awa, awb, ab, wn, bn):
    NBLK = N // MB

    def bodyfn(p0r, p1r, c0r, c1r, xr, ar, br, abr, wnr, bnr, xo, ho):
        cnt = c0r[:, 0:1] + c1r[:, 0:1]
        recip = 1.0 / jnp.maximum(cnt, 1.0)
        aggr = (p0r[:] + p1r[:]) * recip
        xn = jnp.maximum(
            jnp.dot(aggr, ar[:], preferred_element_type=f32)
            + jnp.dot(xr[:], br[:], preferred_element_type=f32) + abr[:], 0.0)
        xo[:] = xn
        ho[:] = jnp.dot(xn, wnr[:], preferred_element_type=f32) + bnr[:]

    return pl.pallas_call(
        bodyfn,
        grid=(NBLK,),
        in_specs=[pl.BlockSpec((MB, D), lambda i: (i, 0)),
                  pl.BlockSpec((MB, D), lambda i: (i + NBLK, 0)),
                  pl.BlockSpec((MB, D), lambda i: (i, 0)),
                  pl.BlockSpec((MB, D), lambda i: (i + NBLK, 0)),
                  pl.BlockSpec((MB, D), lambda i: (i, 0)),
                  pl.BlockSpec((D, D), lambda i: (0, 0)),
                  pl.BlockSpec((D, D), lambda i: (0, 0)),
                  pl.BlockSpec((1, D), lambda i: (0, 0)),
                  pl.BlockSpec((D, D), lambda i: (0, 0)),
                  pl.BlockSpec((1, D), lambda i: (0, 0))],
        out_specs=[pl.BlockSpec((MB, D), lambda i: (i, 0)),
                   pl.BlockSpec((MB, D), lambda i: (i, 0))],
        out_shape=[jax.ShapeDtypeStruct((N, D), f32),
                   jax.ShapeDtypeStruct((N, D), f32)],
    )(S, S, C, C, x, awa, awb, ab, wn, bn)


def _update_cls(S, C, x, awa, awb, ab, w0, b0, w1, b1):
    NBLK = N // MB

    def bodyfn(p0r, p1r, c0r, c1r, xr, ar, br, abr,
               w0r, b0r, w1r, b1r, o_ref, accr):
        i = pl.program_id(0)
        cnt = c0r[:, 0:1] + c1r[:, 0:1]
        recip = 1.0 / jnp.maximum(cnt, 1.0)
        aggr = (p0r[:] + p1r[:]) * recip
        xn = jnp.maximum(
            jnp.dot(aggr, ar[:], preferred_element_type=f32)
            + jnp.dot(xr[:], br[:], preferred_element_type=f32) + abr[:], 0.0)

        @pl.when(i == 0)
        def _():
            accr[:] = jnp.zeros((1, D), f32)

        accr[:] = accr[:] + jnp.sum(xn, axis=0, keepdims=True)

        @pl.when(i == NBLK - 1)
        def _():
            pooled = accr[:] * (1.0 / N)
            z = jnp.maximum(
                jnp.dot(pooled, w0r[:], preferred_element_type=f32) + b0r[:],
                0.0)
            o_ref[:] = jnp.dot(z, w1r[:], preferred_element_type=f32) + b1r[:]

    out = pl.pallas_call(
        bodyfn,
        grid=(NBLK,),
        in_specs=[pl.BlockSpec((MB, D), lambda i: (i, 0)),
                  pl.BlockSpec((MB, D), lambda i: (i + NBLK, 0)),
                  pl.BlockSpec((MB, D), lambda i: (i, 0)),
                  pl.BlockSpec((MB, D), lambda i: (i + NBLK, 0)),
                  pl.BlockSpec((MB, D), lambda i: (i, 0)),
                  pl.BlockSpec((D, D), lambda i: (0, 0)),
                  pl.BlockSpec((D, D), lambda i: (0, 0)),
                  pl.BlockSpec((1, D), lambda i: (0, 0)),
                  pl.BlockSpec((D, D), lambda i: (0, 0)),
                  pl.BlockSpec((1, D), lambda i: (0, 0)),
                  pl.BlockSpec((D, 16), lambda i: (0, 0)),
                  pl.BlockSpec((1, 16), lambda i: (0, 0))],
        out_specs=pl.BlockSpec((1, 16), lambda i: (0, 0)),
        out_shape=jax.ShapeDtypeStruct((1, 16), f32),
        scratch_shapes=[pltpu.VMEM((1, D), f32)],
    )(S, S, C, C, x, awa, awb, ab, w0, b0, w1, b1)
    return out.reshape(16)


def _classifier(h, w0, b0, w1, b1):
    grid = N // MB

    def bodyfn(h_ref, w0r, b0r, w1r, b1r, o_ref, accr):
        i = pl.program_id(0)

        @pl.when(i == 0)
        def _():
            accr[:] = jnp.zeros((1, D), f32)

        accr[:] = accr[:] + jnp.sum(h_ref[:], axis=0, keepdims=True)

        @pl.when(i == grid - 1)
        def _():
            pooled = accr[:] * (1.0 / N)
            z = jnp.maximum(
                jnp.dot(pooled, w0r[:], preferred_element_type=f32) + b0r[:],
                0.0)
            o_ref[:] = jnp.dot(z, w1r[:], preferred_element_type=f32) + b1r[:]

    out = pl.pallas_call(
        bodyfn,
        grid=(grid,),
        in_specs=[pl.BlockSpec((MB, D), lambda i: (i, 0)),
                  pl.BlockSpec((D, D), lambda i: (0, 0)),
                  pl.BlockSpec((1, D), lambda i: (0, 0)),
                  pl.BlockSpec((D, 16), lambda i: (0, 0)),
                  pl.BlockSpec((1, 16), lambda i: (0, 0))],
        out_specs=pl.BlockSpec((1, 16), lambda i: (0, 0)),
        out_shape=jax.ShapeDtypeStruct((1, 16), f32),
        scratch_shapes=[pltpu.VMEM((1, D), f32)],
    )(h, w0, b0, w1, b1)
    return out.reshape(16)


def kernel(x, edge_index, edge_weight,
           lin_W0, lin_b0, agg_W0, agg_b0,
           lin_W1, lin_b1, agg_W1, agg_b1,
           lin_W2, lin_b2, agg_W2, agg_b2,
           cls_W0, cls_b0, cls_W1, cls_b1):
    srcf = edge_index[1]
    dstf = edge_index[0]

    sc_prop = _make_sc_propagate()
    sc_cnt = _make_sc_propagate(with_gather=False)

    C = sc_cnt(x, srcf, dstf, edge_weight)

    h = _mm(x, lin_W0, lin_b0.reshape(1, D))
    S = sc_prop(h, srcf, dstf, edge_weight)
    x1, h1 = _update_mm(S, C, x, agg_W0[:D], agg_W0[D:],
                        agg_b0.reshape(1, D), lin_W1, lin_b1.reshape(1, D))
    S = sc_prop(h1, srcf, dstf, edge_weight)
    x2, h2 = _update_mm(S, C, x1, agg_W1[:D], agg_W1[D:],
                        agg_b1.reshape(1, D), lin_W2, lin_b2.reshape(1, D))
    S = sc_prop(h2, srcf, dstf, edge_weight)
    return _update_cls(S, C, x2, agg_W2[:D], agg_W2[D:],
                       agg_b2.reshape(1, D), cls_W0, cls_b0.reshape(1, D),
                       cls_W1, cls_b1.reshape(1, 16))
